# Initial kernel scaffold; baseline (speedup 1.0000x reference)
import jax
import jax.numpy as jnp
from jax.experimental import pallas as pl


def kernel(x, edge_index, edge_attr, eps1, Wb1_1, bb1_1, Wb2_1, bb2_1, Wm1_1, bm1_1, Wm2_1, bm2_1, gamma1, beta1, eps2, Wb1_2, bb1_2, Wb2_2, bb2_2, Wm1_2, bm1_2, Wm2_2, bm2_2, gamma2, beta2, Wfc1, bfc1, Wfc2, bfc2):
    raise NotImplementedError("write your pallas kernel here")


# speedup vs baseline: 2.4441x; 2.4441x over previous
import functools

import jax
import jax.numpy as jnp
from jax import lax
from jax.experimental import pallas as pl
from jax.experimental.pallas import tpu as pltpu
from jax.experimental.pallas import tpu_sc as plsc

_N = 10000
_E = 320000
_D = 128
_DE = 16
_OUT = 64

_NC = 2
_NS = 16
_NW = _NC * _NS
_EPW = _E // _NW
_C = 80
_NCH = _EPW // _C
_NP = 10240
_RPS = _NP // _NS
_ZR = 128

_EB = 3200


def _edge_body(ea_ref, w11, b11, w21, b21, w12, b12, w22, b22, ee1_ref, ee2_ref):
    ea = ea_ref[...]
    t1 = jnp.maximum(jnp.dot(ea, w11[...], preferred_element_type=jnp.float32) + b11[...], 0.0)
    ee1_ref[...] = jnp.dot(t1, w21[...], preferred_element_type=jnp.float32) + b21[...]
    t2 = jnp.maximum(jnp.dot(ea, w12[...], preferred_element_type=jnp.float32) + b12[...], 0.0)
    ee2_ref[...] = jnp.dot(t2, w22[...], preferred_element_type=jnp.float32) + b22[...]


def _edge_encoder(ea, w11, b11, w21, b21, w12, b12, w22, b22):
    wspec = lambda a: pl.BlockSpec(a.shape, lambda i: tuple(0 for _ in a.shape))
    args = (w11, b11, w21, b21, w12, b12, w22, b22)
    return pl.pallas_call(
        _edge_body,
        grid=(_E // _EB,),
        in_specs=[pl.BlockSpec((_EB, _DE), lambda i: (i, 0))] + [wspec(a) for a in args],
        out_specs=[pl.BlockSpec((_EB, _D), lambda i: (i, 0))] * 2,
        out_shape=[jax.ShapeDtypeStruct((_E, _D), jnp.float32)] * 2,
    )(ea, *args)


def _node_body(x_ref, p_ref, eps_ref, wm1, bm1, wm2, bm2, g_ref, b_ref, out_ref):
    agg = p_ref[0, :_N] + p_ref[1, :_N]
    h = x_ref[...] * (1.0 + eps_ref[...]) + agg
    t = jnp.maximum(jnp.dot(h, wm1[...], preferred_element_type=jnp.float32) + bm1[...], 0.0)
    z = jnp.dot(t, wm2[...], preferred_element_type=jnp.float32) + bm2[...]
    z = jnp.maximum(z, 0.0)
    m = jnp.mean(z, axis=0, keepdims=True)
    v = jnp.mean((z - m) * (z - m), axis=0, keepdims=True)
    out_ref[...] = (z - m) * lax.rsqrt(v + 1e-5) * g_ref[...] + b_ref[...]


def _node_update(x, parts, eps, wm1, bm1, wm2, bm2, gamma, beta):
    return pl.pallas_call(
        _node_body,
        out_shape=jax.ShapeDtypeStruct((_N, _D), jnp.float32),
    )(x, parts, eps, wm1, bm1, wm2, bm2, gamma, beta)


def _readout_body(h1_ref, h2_ref, wa, wb, b1, w2, b2, out_ref):
    xf = (jnp.dot(h1_ref[...], wa[...], preferred_element_type=jnp.float32)
          + jnp.dot(h2_ref[...], wb[...], preferred_element_type=jnp.float32) + b1[...])
    xf = jnp.maximum(xf, 0.0)
    out_ref[...] = jnp.dot(xf, w2[...], preferred_element_type=jnp.float32) + b2[...]


def _readout(h1, h2, wfc1, bfc1, wfc2, bfc2):
    return pl.pallas_call(
        _readout_body,
        out_shape=jax.ShapeDtypeStruct((_N, _OUT), jnp.float32),
    )(h1, h2, wfc1[:_D], wfc1[_D:], bfc1, wfc2, bfc2)


def _sc_body(table, ee, src, dst, parts, src_v, dst_v, rows_v, ee_v, zero_v, acc_sh, sem):
    c = lax.axis_index("c")
    s = lax.axis_index("s")
    wid = c * _NS + s

    def zfill(i, _):
        for j in range(_D // 16):
            zero_v[i, pl.ds(j * 16, 16)] = jnp.zeros((16,), jnp.float32)
        return 0

    lax.fori_loop(0, _ZR, zfill, 0)

    def zcopy(k, _):
        pltpu.sync_copy(zero_v, acc_sh.at[pl.ds(s * _RPS + k * _ZR, _ZR)])
        return 0

    lax.fori_loop(0, _RPS // _ZR, zcopy, 0)
    plsc.subcore_barrier()

    ebase = wid * _EPW

    def chunk(k, _):
        base = ebase + k * _C
        pltpu.sync_copy(src.at[pl.ds(base, _C)], src_v)
        pltpu.sync_copy(dst.at[pl.ds(base, _C)], dst_v.at[0])
        pltpu.async_copy(table.at[src_v], rows_v, sem).wait()
        pltpu.sync_copy(ee.at[pl.ds(base, _C)], ee_v)

        def combine(i, _):
            for j in range(_D // 16):
                sl = pl.ds(j * 16, 16)
                rows_v[i, sl] = jnp.maximum(rows_v[i, sl] + ee_v[i, sl], 0.0)
            return 0

        lax.fori_loop(0, _C, combine, 0)
        pltpu.sync_copy(rows_v, acc_sh.at[dst_v.at[0]], add=True)
        return 0

    lax.fori_loop(0, _NCH, chunk, 0)
    plsc.subcore_barrier()
    pltpu.sync_copy(acc_sh.at[pl.ds(s * _RPS, _RPS)],
                    parts.at[c, pl.ds(s * _RPS, _RPS)])


_sc_gather_scatter = functools.partial(
    pl.kernel,
    out_type=jax.ShapeDtypeStruct((_NC, _NP, _D), jnp.float32),
    mesh=plsc.VectorSubcoreMesh(core_axis_name="c", subcore_axis_name="s"),
    scratch_types=[
        pltpu.VMEM((_C,), jnp.int32),
        pltpu.VMEM((1, _C), jnp.int32),
        pltpu.VMEM((_C, _D), jnp.float32),
        pltpu.VMEM((_C, _D), jnp.float32),
        pltpu.VMEM((_ZR, _D), jnp.float32),
        pltpu.VMEM_SHARED((_NP, _D), jnp.float32),
        pltpu.SemaphoreType.DMA,
    ],
)(_sc_body)


def kernel(x, edge_index, edge_attr,
           eps1, Wb1_1, bb1_1, Wb2_1, bb2_1, Wm1_1, bm1_1, Wm2_1, bm2_1, gamma1, beta1,
           eps2, Wb1_2, bb1_2, Wb2_2, bb2_2, Wm1_2, bm1_2, Wm2_2, bm2_2, gamma2, beta2,
           Wfc1, bfc1, Wfc2, bfc2):
    src = edge_index[0]
    dst = edge_index[1]
    r = lambda a: a.reshape(1, -1)
    ee1, ee2 = _edge_encoder(edge_attr, Wb1_1, r(bb1_1), Wb2_1, r(bb2_1),
                             Wb1_2, r(bb1_2), Wb2_2, r(bb2_2))
    parts1 = _sc_gather_scatter(x, ee1, src, dst)
    h1 = _node_update(x, parts1, eps1.reshape(1, 1), Wm1_1, r(bm1_1), Wm2_1,
                      r(bm2_1), r(gamma1), r(beta1))
    parts2 = _sc_gather_scatter(h1, ee2, src, dst)
    h2 = _node_update(h1, parts2, eps2.reshape(1, 1), Wm1_2, r(bm1_2), Wm2_2,
                      r(bm2_2), r(gamma2), r(beta2))
    return _readout(h1, h2, Wfc1, r(bfc1), Wfc2, r(bfc2))


# speedup vs baseline: 4.3316x; 1.7723x over previous
import functools

import jax
import jax.numpy as jnp
from jax import lax
from jax.experimental import pallas as pl
from jax.experimental.pallas import tpu as pltpu
from jax.experimental.pallas import tpu_sc as plsc

_N = 10000
_E = 320000
_D = 128
_DE = 16
_OUT = 64

_NC = 2
_NS = 16
_NW = _NC * _NS
_EPW = _E // _NW
_C = 80
_NCH = _EPW // _C
_NP = 10240
_RPS = _NP // _NS

_EB = 3200


def _edge_body(ea_ref, w11, b11, w21, b21, w12, b12, w22, b22, ee1_ref, ee2_ref):
    ea = ea_ref[...]
    t1 = jnp.maximum(jnp.dot(ea, w11[...], preferred_element_type=jnp.float32) + b11[...], 0.0)
    ee1_ref[...] = jnp.dot(t1, w21[...], preferred_element_type=jnp.float32) + b21[...]
    t2 = jnp.maximum(jnp.dot(ea, w12[...], preferred_element_type=jnp.float32) + b12[...], 0.0)
    ee2_ref[...] = jnp.dot(t2, w22[...], preferred_element_type=jnp.float32) + b22[...]


def _edge_encoder(ea, w11, b11, w21, b21, w12, b12, w22, b22):
    wspec = lambda a: pl.BlockSpec(a.shape, lambda i: tuple(0 for _ in a.shape))
    args = (w11, b11, w21, b21, w12, b12, w22, b22)
    return pl.pallas_call(
        _edge_body,
        grid=(_E // _EB,),
        in_specs=[pl.BlockSpec((_EB, _DE), lambda i: (i, 0))] + [wspec(a) for a in args],
        out_specs=[pl.BlockSpec((_EB, _D), lambda i: (i, 0))] * 2,
        out_shape=[jax.ShapeDtypeStruct((_E, _D), jnp.float32)] * 2,
    )(ea, *args)


def _node_body(x_ref, p_ref, eps_ref, wm1, bm1, wm2, bm2, g_ref, b_ref, out_ref):
    agg = p_ref[0, :_N] + p_ref[1, :_N]
    h = x_ref[...] * (1.0 + eps_ref[...]) + agg
    t = jnp.maximum(jnp.dot(h, wm1[...], preferred_element_type=jnp.float32) + bm1[...], 0.0)
    z = jnp.dot(t, wm2[...], preferred_element_type=jnp.float32) + bm2[...]
    z = jnp.maximum(z, 0.0)
    m = jnp.mean(z, axis=0, keepdims=True)
    v = jnp.mean((z - m) * (z - m), axis=0, keepdims=True)
    out_ref[...] = (z - m) * lax.rsqrt(v + 1e-5) * g_ref[...] + b_ref[...]


def _node_update(x, parts, eps, wm1, bm1, wm2, bm2, gamma, beta):
    return pl.pallas_call(
        _node_body,
        out_shape=jax.ShapeDtypeStruct((_N, _D), jnp.float32),
    )(x, parts, eps, wm1, bm1, wm2, bm2, gamma, beta)


def _readout_body(h1_ref, h2_ref, wa, wb, b1, w2, b2, out_ref):
    xf = (jnp.dot(h1_ref[...], wa[...], preferred_element_type=jnp.float32)
          + jnp.dot(h2_ref[...], wb[...], preferred_element_type=jnp.float32) + b1[...])
    xf = jnp.maximum(xf, 0.0)
    out_ref[...] = jnp.dot(xf, w2[...], preferred_element_type=jnp.float32) + b2[...]


def _readout(h1, h2, wfc1, bfc1, wfc2, bfc2):
    return pl.pallas_call(
        _readout_body,
        out_shape=jax.ShapeDtypeStruct((_N, _OUT), jnp.float32),
    )(h1, h2, wfc1[:_D], wfc1[_D:], bfc1, wfc2, bfc2)


def _sc_body(table, ee, src, dst, parts, srcb, dstb, rows_v, ee_v, acc_sh,
             isem0, isem1, gsem0, gsem1, esem0, esem1):
    c = lax.axis_index("c")
    s = lax.axis_index("s")
    wid = c * _NS + s
    isem = (isem0, isem1)
    gsem = (gsem0, gsem1)
    esem = (esem0, esem1)

    def zfill(i, _):
        for j in range(_D // 16):
            rows_v[0, i, pl.ds(j * 16, 16)] = jnp.zeros((16,), jnp.float32)
        return 0

    lax.fori_loop(0, _C, zfill, 0)

    def zcopy(k, _):
        pltpu.sync_copy(rows_v.at[0], acc_sh.at[pl.ds(s * _RPS + k * _C, _C)])
        return 0

    lax.fori_loop(0, _RPS // _C, zcopy, 0)
    plsc.subcore_barrier()

    ebase = wid * _EPW

    def issue_idx(k, b):
        base = ebase + k * _C
        pltpu.async_copy(src.at[pl.ds(base, _C)], srcb.at[b], isem[b])
        pltpu.async_copy(dst.at[pl.ds(base, _C)], dstb.at[b], isem[b])

    def wait_idx(b):
        pltpu.make_async_copy(src.at[pl.ds(0, _C)], srcb.at[b], isem[b]).wait()
        pltpu.make_async_copy(dst.at[pl.ds(0, _C)], dstb.at[b], isem[b]).wait()

    def issue_data(k, b):
        pltpu.async_copy(table.at[srcb.at[b]], rows_v.at[b], gsem[b])
        pltpu.async_copy(ee.at[pl.ds(ebase + k * _C, _C)], ee_v.at[b], esem[b])

    def wait_data(b):
        pltpu.make_async_copy(ee.at[pl.ds(0, _C)], rows_v.at[b], gsem[b]).wait()
        pltpu.make_async_copy(ee.at[pl.ds(0, _C)], ee_v.at[b], esem[b]).wait()

    def process(b):
        def combine(i, _):
            for j in range(_D // 16):
                sl = pl.ds(j * 16, 16)
                rows_v[b, i, sl] = jnp.maximum(rows_v[b, i, sl] + ee_v[b, i, sl], 0.0)
            return 0

        lax.fori_loop(0, _C, combine, 0)
        pltpu.sync_copy(rows_v.at[b], acc_sh.at[dstb.at[b]], add=True)

    issue_idx(0, 0)
    wait_idx(0)
    issue_data(0, 0)
    issue_idx(1, 1)

    def pair(i, _):
        for b in (0, 1):
            k = 2 * i + b
            nb = 1 - b
            wait_idx(nb)
            issue_data(k + 1, nb)
            wait_data(b)
            process(b)
            issue_idx(k + 2, b)
        return 0

    lax.fori_loop(0, (_NCH - 1) // 2, pair, 0)
    wait_data(0)
    process(0)
    wait_idx(1)

    plsc.subcore_barrier()
    pltpu.sync_copy(acc_sh.at[pl.ds(s * _RPS, _RPS)],
                    parts.at[c, pl.ds(s * _RPS, _RPS)])


_sc_gather_scatter = functools.partial(
    pl.kernel,
    out_type=jax.ShapeDtypeStruct((_NC, _NP, _D), jnp.float32),
    mesh=plsc.VectorSubcoreMesh(core_axis_name="c", subcore_axis_name="s"),
    scratch_types=[
        pltpu.VMEM((2, _C), jnp.int32),
        pltpu.VMEM((2, _C), jnp.int32),
        pltpu.VMEM((2, _C, _D), jnp.float32),
        pltpu.VMEM((2, _C, _D), jnp.float32),
        pltpu.VMEM_SHARED((_NP, _D), jnp.float32),
        pltpu.SemaphoreType.DMA,
        pltpu.SemaphoreType.DMA,
        pltpu.SemaphoreType.DMA,
        pltpu.SemaphoreType.DMA,
        pltpu.SemaphoreType.DMA,
        pltpu.SemaphoreType.DMA,
    ],
)(_sc_body)


def kernel(x, edge_index, edge_attr,
           eps1, Wb1_1, bb1_1, Wb2_1, bb2_1, Wm1_1, bm1_1, Wm2_1, bm2_1, gamma1, beta1,
           eps2, Wb1_2, bb1_2, Wb2_2, bb2_2, Wm1_2, bm1_2, Wm2_2, bm2_2, gamma2, beta2,
           Wfc1, bfc1, Wfc2, bfc2):
    pad = jnp.zeros((1, _C), jnp.int32)
    ei = jnp.concatenate([edge_index, jnp.broadcast_to(pad, (2, _C))], axis=1)
    src = ei[0]
    dst = ei[1]
    r = lambda a: a.reshape(1, -1)
    ee1, ee2 = _edge_encoder(edge_attr, Wb1_1, r(bb1_1), Wb2_1, r(bb2_1),
                             Wb1_2, r(bb1_2), Wb2_2, r(bb2_2))
    parts1 = _sc_gather_scatter(x, ee1, src, dst)
    h1 = _node_update(x, parts1, eps1.reshape(1, 1), Wm1_1, r(bm1_1), Wm2_1,
                      r(bm2_1), r(gamma1), r(beta1))
    parts2 = _sc_gather_scatter(h1, ee2, src, dst)
    h2 = _node_update(h1, parts2, eps2.reshape(1, 1), Wm1_2, r(bm1_2), Wm2_2,
                      r(bm2_2), r(gamma2), r(beta2))
    return _readout(h1, h2, Wfc1, r(bfc1), Wfc2, r(bfc2))


# speedup vs baseline: 4.3412x; 1.0022x over previous
import functools

import jax
import jax.numpy as jnp
from jax import lax
from jax.experimental import pallas as pl
from jax.experimental.pallas import tpu as pltpu
from jax.experimental.pallas import tpu_sc as plsc

_N = 10000
_E = 320000
_D = 128
_DE = 16
_OUT = 64

_NC = 2
_NS = 16
_NW = _NC * _NS
_EPW = _E // _NW
_C = 80
_NCH = _EPW // _C
_NP = 10240
_RPS = _NP // _NS

_EB = 3200


def _edge_body(ea_ref, w1, b1, w2, b2, ee_ref):
    ea = ea_ref[...]
    t = jnp.maximum(jnp.dot(ea, w1[...], preferred_element_type=jnp.float32) + b1[...], 0.0)
    ee_ref[...] = jnp.dot(t, w2[...], preferred_element_type=jnp.float32) + b2[...]


def _edge_encoder(ea, w1, b1, w2, b2):
    wspec = lambda a: pl.BlockSpec(a.shape, lambda i: tuple(0 for _ in a.shape))
    args = (w1, b1, w2, b2)
    return pl.pallas_call(
        _edge_body,
        grid=(_E // _EB,),
        in_specs=[pl.BlockSpec((_EB, _DE), lambda i: (i, 0))] + [wspec(a) for a in args],
        out_specs=pl.BlockSpec((_EB, _D), lambda i: (i, 0)),
        out_shape=jax.ShapeDtypeStruct((_E, _D), jnp.float32),
    )(ea, *args)


def _node_body(x_ref, p_ref, eps_ref, wm1, bm1, wm2, bm2, g_ref, b_ref, out_ref):
    agg = p_ref[0, :_N] + p_ref[1, :_N]
    h = x_ref[...] * (1.0 + eps_ref[...]) + agg
    t = jnp.maximum(jnp.dot(h, wm1[...], preferred_element_type=jnp.float32) + bm1[...], 0.0)
    z = jnp.dot(t, wm2[...], preferred_element_type=jnp.float32) + bm2[...]
    z = jnp.maximum(z, 0.0)
    m = jnp.mean(z, axis=0, keepdims=True)
    v = jnp.mean((z - m) * (z - m), axis=0, keepdims=True)
    out_ref[...] = (z - m) * lax.rsqrt(v + 1e-5) * g_ref[...] + b_ref[...]


def _node_update(x, parts, eps, wm1, bm1, wm2, bm2, gamma, beta):
    return pl.pallas_call(
        _node_body,
        out_shape=jax.ShapeDtypeStruct((_N, _D), jnp.float32),
    )(x, parts, eps, wm1, bm1, wm2, bm2, gamma, beta)


def _readout_body(h1_ref, h2_ref, wa, wb, b1, w2, b2, out_ref):
    xf = (jnp.dot(h1_ref[...], wa[...], preferred_element_type=jnp.float32)
          + jnp.dot(h2_ref[...], wb[...], preferred_element_type=jnp.float32) + b1[...])
    xf = jnp.maximum(xf, 0.0)
    out_ref[...] = jnp.dot(xf, w2[...], preferred_element_type=jnp.float32) + b2[...]


def _readout(h1, h2, wfc1, bfc1, wfc2, bfc2):
    return pl.pallas_call(
        _readout_body,
        out_shape=jax.ShapeDtypeStruct((_N, _OUT), jnp.float32),
    )(h1, h2, wfc1[:_D], wfc1[_D:], bfc1, wfc2, bfc2)


def _sc_body(table, ee, src, dst, parts, idxb, rows_v, ee_v, acc_sh,
             isem0, isem1, gsem0, gsem1, esem0, esem1):
    c = lax.axis_index("c")
    s = lax.axis_index("s")
    wid = c * _NS + s
    isem = (isem0, isem1)
    gsem = (gsem0, gsem1)
    esem = (esem0, esem1)

    def zfill(i, _):
        for j in range(_D // 16):
            rows_v[0, i, pl.ds(j * 16, 16)] = jnp.zeros((16,), jnp.float32)
        return 0

    lax.fori_loop(0, _C, zfill, 0)

    def zcopy(k, _):
        pltpu.sync_copy(rows_v.at[0], acc_sh.at[pl.ds(s * _RPS + k * _C, _C)])
        return 0

    lax.fori_loop(0, _RPS // _C, zcopy, 0)
    plsc.subcore_barrier()

    ebase = wid * _EPW

    def issue_idx(k, b):
        base = ebase + k * _C
        pltpu.async_copy(src.at[pl.ds(base, _C)], idxb.at[b, 0], isem[b])
        pltpu.async_copy(dst.at[pl.ds(base, _C)], idxb.at[b, 1], isem[b])

    def wait_idx(b):
        pltpu.make_async_copy(src.at[pl.ds(0, _C)], idxb.at[b, 0], isem[b]).wait()
        pltpu.make_async_copy(dst.at[pl.ds(0, _C)], idxb.at[b, 1], isem[b]).wait()

    def issue_data(k, b):
        pltpu.async_copy(table.at[idxb.at[b, 0]], rows_v.at[b], gsem[b])
        pltpu.async_copy(ee.at[pl.ds(ebase + k * _C, _C)], ee_v.at[b], esem[b])

    def wait_data(b):
        pltpu.make_async_copy(ee.at[pl.ds(0, _C)], rows_v.at[b], gsem[b]).wait()
        pltpu.make_async_copy(ee.at[pl.ds(0, _C)], ee_v.at[b], esem[b]).wait()

    def process(b):
        def combine(i, _):
            for j in range(_D // 16):
                sl = pl.ds(j * 16, 16)
                rows_v[b, i, sl] = jnp.maximum(rows_v[b, i, sl] + ee_v[b, i, sl], 0.0)
            return 0

        lax.fori_loop(0, _C, combine, 0)
        pltpu.sync_copy(rows_v.at[b], acc_sh.at[idxb.at[b, 1]], add=True)

    issue_idx(0, 0)
    wait_idx(0)
    issue_data(0, 0)
    issue_idx(1, 1)

    def pair(i, _):
        for b in (0, 1):
            k = 2 * i + b
            nb = 1 - b
            wait_idx(nb)
            issue_data(k + 1, nb)
            wait_data(b)
            process(b)
            issue_idx(k + 2, b)
        return 0

    lax.fori_loop(0, (_NCH - 1) // 2, pair, 0)
    wait_data(0)
    process(0)
    wait_idx(1)

    plsc.subcore_barrier()
    pltpu.sync_copy(acc_sh.at[pl.ds(s * _RPS, _RPS)],
                    parts.at[c, pl.ds(s * _RPS, _RPS)])


_sc_gather_scatter = functools.partial(
    pl.kernel,
    out_type=jax.ShapeDtypeStruct((_NC, _NP, _D), jnp.float32),
    mesh=plsc.VectorSubcoreMesh(core_axis_name="c", subcore_axis_name="s"),
    scratch_types=[
        pltpu.VMEM((2, 2, _C), jnp.int32),
        pltpu.VMEM((2, _C, _D), jnp.float32),
        pltpu.VMEM((2, _C, _D), jnp.float32),
        pltpu.VMEM_SHARED((_NP, _D), jnp.float32),
        pltpu.SemaphoreType.DMA,
        pltpu.SemaphoreType.DMA,
        pltpu.SemaphoreType.DMA,
        pltpu.SemaphoreType.DMA,
        pltpu.SemaphoreType.DMA,
        pltpu.SemaphoreType.DMA,
    ],
)(_sc_body)


def kernel(x, edge_index, edge_attr,
           eps1, Wb1_1, bb1_1, Wb2_1, bb2_1, Wm1_1, bm1_1, Wm2_1, bm2_1, gamma1, beta1,
           eps2, Wb1_2, bb1_2, Wb2_2, bb2_2, Wm1_2, bm1_2, Wm2_2, bm2_2, gamma2, beta2,
           Wfc1, bfc1, Wfc2, bfc2):
    pad = jnp.zeros((1, _C), jnp.int32)
    ei = jnp.concatenate([edge_index, jnp.broadcast_to(pad, (2, _C))], axis=1)
    r = lambda a: a.reshape(1, -1)
    ee1 = _edge_encoder(edge_attr, Wb1_1, r(bb1_1), Wb2_1, r(bb2_1))
    ee2 = _edge_encoder(edge_attr, Wb1_2, r(bb1_2), Wb2_2, r(bb2_2))
    parts1 = _sc_gather_scatter(x, ee1, ei[0], ei[1])
    h1 = _node_update(x, parts1, eps1.reshape(1, 1), Wm1_1, r(bm1_1), Wm2_1,
                      r(bm2_1), r(gamma1), r(beta1))
    parts2 = _sc_gather_scatter(h1, ee2, ei[0], ei[1])
    h2 = _node_update(h1, parts2, eps2.reshape(1, 1), Wm1_2, r(bm1_2), Wm2_2,
                      r(bm2_2), r(gamma2), r(beta2))
    return _readout(h1, h2, Wfc1, r(bfc1), Wfc2, r(bfc2))
